# baseline (speedup 1.0000x reference)
import functools

import jax
import jax.numpy as jnp
from jax import lax
from jax.experimental import pallas as pl
from jax.experimental.pallas import tpu as pltpu
from jax.experimental.pallas import tpu_sc as plsc

HEIGHT = 512
WIDTH = 512
INTENSITY = 255.0
NBINS = HEIGHT * WIDTH
HIST_WORDS = 2 * NBINS

N_POINTS = 2_000_000
CHUNK = 3200
KROWS = CHUNK // 128
NCHUNKS = N_POINTS // CHUNK
NC, NS = 2, 16
NW = NC * NS

MM_BLOCK = 20_480
MM_GRID = -(-N_POINTS // MM_BLOCK)

STRIPE = HIST_WORDS // NS
DUMP = NBINS // NS


def _minmax_body(pc_ref, o_ref, fl_ref):
    i = pl.program_id(0)
    d = pc_ref[:]
    d8 = jnp.concatenate(
        [d, jnp.zeros((5, MM_BLOCK), jnp.float32)], axis=0)
    fl_ref[:] = jnp.transpose(
        d8.reshape(8, MM_BLOCK // 128, 128), (1, 0, 2))
    cid = lax.broadcasted_iota(jnp.int32, (3, MM_BLOCK), 1)
    valid = (i * MM_BLOCK + cid) < N_POINTS
    dlo = jnp.where(valid, d, jnp.inf)
    dhi = jnp.where(valid, d, -jnp.inf)
    dmin = jnp.broadcast_to(jnp.min(dlo, axis=1, keepdims=True), (3, 128))
    dmax = jnp.broadcast_to(jnp.max(dhi, axis=1, keepdims=True), (3, 128))
    pad1 = jnp.zeros((1, 128), jnp.float32)
    cur = jnp.concatenate([dmin, pad1, dmax, pad1], axis=0)
    rows = lax.broadcasted_iota(jnp.int32, (8, 128), 0)
    acc = o_ref[:]
    comb = jnp.where(rows < 4, jnp.minimum(acc, cur), jnp.maximum(acc, cur))
    o_ref[:] = jnp.where(i == 0, cur, comb)


_minmax_call = pl.pallas_call(
    _minmax_body,
    grid=(MM_GRID,),
    in_specs=[pl.BlockSpec((3, MM_BLOCK), lambda i: (0, i))],
    out_specs=[pl.BlockSpec((8, 128), lambda i: (0, 0)),
               pl.BlockSpec((MM_BLOCK // 128, 8, 128), lambda i: (i, 0, 0))],
    out_shape=[jax.ShapeDtypeStruct((8, 128), jnp.float32),
               jax.ShapeDtypeStruct(
                   (MM_GRID * MM_BLOCK // 128, 8, 128), jnp.float32)],
)


def _combine_body(p_ref, o_ref):
    s = p_ref[0, 0] + p_ref[1, 0]
    c = p_ref[0, 1] + p_ref[1, 1]
    o_ref[:] = jnp.where(c > 0, s / c, 0.0)


_combine_call = pl.pallas_call(
    _combine_body,
    grid=(8,),
    in_specs=[pl.BlockSpec((2, 2, 64, 512), lambda r: (0, 0, r, 0))],
    out_specs=pl.BlockSpec((64, 512), lambda r: (r, 0)),
    out_shape=jax.ShapeDtypeStruct((HEIGHT, WIDTH), jnp.float32),
)


NPAIRS = (NCHUNKS // NW + 1 + 1) // 2


def _sc_body(pc_ref, mm_ref, out_ref,
             pcbuf, mmbuf, idxsa, idxca, zssa, idxsb, idxcb, zssb,
             ones, obuf, hist, sema, semb):
    cid = lax.axis_index("c")
    sid = lax.axis_index("s")
    wid = sid * NC + cid

    def zero_obuf(t, _):
        obuf[pl.ds(t * 16, 16)] = jnp.zeros((16,), jnp.float32)
        return 0
    lax.fori_loop(0, DUMP // 16, zero_obuf, 0)
    pltpu.sync_copy(obuf, hist.at[pl.ds(sid * STRIPE, DUMP)])
    pltpu.sync_copy(obuf, hist.at[pl.ds(sid * STRIPE + DUMP, DUMP)])

    pltpu.sync_copy(mm_ref, mmbuf)
    xmin = mmbuf[pl.ds(0, 16)]
    ymin = mmbuf[pl.ds(128, 16)]
    zmin = mmbuf[pl.ds(256, 16)]
    xmax = mmbuf[pl.ds(512, 16)]
    ymax = mmbuf[pl.ds(640, 16)]
    zmax = mmbuf[pl.ds(768, 16)]
    rx = xmax - xmin
    ry = ymax - ymin
    rz = zmax - zmin

    def init_ones(t, _):
        ones[pl.ds(t * 16, 16)] = jnp.full((16,), 1.0, jnp.float32)
        return 0
    lax.fori_loop(0, CHUNK // 16, init_ones, 0)

    plsc.subcore_barrier()

    dummy = out_ref.at[cid, 0, pl.ds(0, CHUNK)]

    def compute_chunk(g, idxs, idxc, zss):
        pltpu.sync_copy(pc_ref.at[pl.ds(g * KROWS, KROWS)], pcbuf)

        def row_body(j, _):
            o128 = j * 128
            for u in range(8):
                xv = pcbuf[j, 0, pl.ds(u * 16, 16)]
                yv = pcbuf[j, 1, pl.ds(u * 16, 16)]
                zv = pcbuf[j, 2, pl.ds(u * 16, 16)]
                xb = ((xv - xmin) * (WIDTH - 1.0) / rx).astype(jnp.int32)
                yb = ((yv - ymin) * (HEIGHT - 1.0) / ry).astype(jnp.int32)
                idx = (511 - yb) * 512 + xb
                idx = jnp.minimum(jnp.maximum(idx, 0), NBINS - 1)
                zs = (zv - zmin) * INTENSITY / rz
                o = o128 + u * 16
                idxs[pl.ds(o, 16)] = idx
                idxc[pl.ds(o, 16)] = idx + NBINS
                zss[pl.ds(o, 16)] = zs
            return 0
        lax.fori_loop(0, KROWS, row_body, 0)

    def pair_body(p, _):
        g0 = wid + (2 * p) * NW
        g1 = g0 + NW

        @pl.when(g0 < NCHUNKS)
        def _():
            @pl.when(p > 0)
            def _():
                pltpu.make_async_copy(dummy, zssa, sema).wait()
                pltpu.make_async_copy(dummy, zssa, sema).wait()
            compute_chunk(g0, idxsa, idxca, zssa)
            pltpu.async_copy(zssa, hist.at[idxsa], sema, add=True)
            pltpu.async_copy(ones, hist.at[idxca], sema, add=True)

        @pl.when(g1 < NCHUNKS)
        def _():
            @pl.when(p > 0)
            def _():
                pltpu.make_async_copy(dummy, zssb, semb).wait()
                pltpu.make_async_copy(dummy, zssb, semb).wait()
            compute_chunk(g1, idxsb, idxcb, zssb)
            pltpu.async_copy(zssb, hist.at[idxsb], semb, add=True)
            pltpu.async_copy(ones, hist.at[idxcb], semb, add=True)
        return 0
    lax.fori_loop(0, NPAIRS, pair_body, 0)

    pltpu.make_async_copy(dummy, zssa, sema).wait()
    pltpu.make_async_copy(dummy, zssa, sema).wait()
    pltpu.make_async_copy(dummy, zssb, semb).wait()
    pltpu.make_async_copy(dummy, zssb, semb).wait()

    plsc.subcore_barrier()

    pltpu.sync_copy(hist.at[pl.ds(sid * DUMP, DUMP)], obuf)
    pltpu.sync_copy(obuf, out_ref.at[cid, 0, pl.ds(sid * DUMP, DUMP)])
    pltpu.sync_copy(hist.at[pl.ds(NBINS + sid * DUMP, DUMP)], obuf)
    pltpu.sync_copy(obuf, out_ref.at[cid, 1, pl.ds(sid * DUMP, DUMP)])


_sc_call = pl.kernel(
    _sc_body,
    out_type=jax.ShapeDtypeStruct((NC, 2, NBINS), jnp.float32),
    mesh=plsc.VectorSubcoreMesh(core_axis_name="c", subcore_axis_name="s",
                                num_cores=NC, num_subcores=NS),
    scratch_types=[
        pltpu.VMEM((KROWS, 8, 128), jnp.float32),
        pltpu.VMEM((1024,), jnp.float32),
        pltpu.VMEM((CHUNK,), jnp.int32),
        pltpu.VMEM((CHUNK,), jnp.int32),
        pltpu.VMEM((CHUNK,), jnp.float32),
        pltpu.VMEM((CHUNK,), jnp.int32),
        pltpu.VMEM((CHUNK,), jnp.int32),
        pltpu.VMEM((CHUNK,), jnp.float32),
        pltpu.VMEM((CHUNK,), jnp.float32),
        pltpu.VMEM((DUMP,), jnp.float32),
        pltpu.VMEM_SHARED((HIST_WORDS,), jnp.float32),
        pltpu.SemaphoreType.DMA,
        pltpu.SemaphoreType.DMA,
    ],
    compiler_params=pltpu.CompilerParams(needs_layout_passes=False),
)


@jax.jit
def kernel(pc):
    mm, pcf = _minmax_call(pc.T)
    parts = _sc_call(pcf, mm.reshape(-1))
    return _combine_call(parts.reshape(NC, 2, HEIGHT, WIDTH))


# speedup vs baseline: 1.1460x; 1.1460x over previous
import functools

import jax
import jax.numpy as jnp
from jax import lax
from jax.experimental import pallas as pl
from jax.experimental.pallas import tpu as pltpu
from jax.experimental.pallas import tpu_sc as plsc

HEIGHT = 512
WIDTH = 512
INTENSITY = 255.0
NBINS = HEIGHT * WIDTH
HIST_WORDS = 2 * NBINS

N_POINTS = 2_000_000
CHUNK = 3200
KROWS = CHUNK // 128
NCHUNKS = N_POINTS // CHUNK
NC, NS = 2, 16
NW = NC * NS

MM_BLOCK = 40_960
MM_GRID = -(-N_POINTS // MM_BLOCK)

STRIPE = HIST_WORDS // NS
DUMP = NBINS // NS


def _minmax_body(pc_ref, o_ref, fl_ref):
    i = pl.program_id(0)
    d = pc_ref[:]
    d8 = jnp.concatenate(
        [d, jnp.zeros((5, MM_BLOCK), jnp.float32)], axis=0)
    fl_ref[:] = jnp.transpose(
        d8.reshape(8, MM_BLOCK // 128, 128), (1, 0, 2))
    cid = lax.broadcasted_iota(jnp.int32, (3, MM_BLOCK), 1)
    valid = (i * MM_BLOCK + cid) < N_POINTS
    dlo = jnp.where(valid, d, jnp.inf)
    dhi = jnp.where(valid, d, -jnp.inf)
    dmin = jnp.broadcast_to(jnp.min(dlo, axis=1, keepdims=True), (3, 128))
    dmax = jnp.broadcast_to(jnp.max(dhi, axis=1, keepdims=True), (3, 128))
    pad1 = jnp.zeros((1, 128), jnp.float32)
    cur = jnp.concatenate([dmin, pad1, dmax, pad1], axis=0)
    rows = lax.broadcasted_iota(jnp.int32, (8, 128), 0)
    acc = o_ref[:]
    comb = jnp.where(rows < 4, jnp.minimum(acc, cur), jnp.maximum(acc, cur))
    o_ref[:] = jnp.where(i == 0, cur, comb)


_minmax_call = pl.pallas_call(
    _minmax_body,
    grid=(MM_GRID,),
    in_specs=[pl.BlockSpec((3, MM_BLOCK), lambda i: (0, i))],
    out_specs=[pl.BlockSpec((8, 128), lambda i: (0, 0)),
               pl.BlockSpec((MM_BLOCK // 128, 8, 128), lambda i: (i, 0, 0))],
    out_shape=[jax.ShapeDtypeStruct((8, 128), jnp.float32),
               jax.ShapeDtypeStruct(
                   (MM_GRID * MM_BLOCK // 128, 8, 128), jnp.float32)],
)


def _combine_body(p_ref, o_ref):
    s = p_ref[0, 0] + p_ref[1, 0]
    c = p_ref[0, 1] + p_ref[1, 1]
    o_ref[:] = jnp.where(c > 0, s / c, 0.0)


_combine_call = pl.pallas_call(
    _combine_body,
    grid=(8,),
    in_specs=[pl.BlockSpec((2, 2, 64, 512), lambda r: (0, 0, r, 0))],
    out_specs=pl.BlockSpec((64, 512), lambda r: (r, 0)),
    out_shape=jax.ShapeDtypeStruct((HEIGHT, WIDTH), jnp.float32),
)


NPAIRS = (NCHUNKS // NW + 1 + 1) // 2


def _sc_body(pc_ref, mm_ref, out_ref,
             pcbuf, pcbufb, mmbuf, idxsa, idxca, zssa, idxsb, idxcb, zssb,
             ones, obuf, hist, sema, semb, semia, semib):
    cid = lax.axis_index("c")
    sid = lax.axis_index("s")
    wid = sid * NC + cid

    def zero_obuf(t, _):
        obuf[pl.ds(t * 16, 16)] = jnp.zeros((16,), jnp.float32)
        return 0
    lax.fori_loop(0, DUMP // 16, zero_obuf, 0)
    pltpu.sync_copy(obuf, hist.at[pl.ds(sid * STRIPE, DUMP)])
    pltpu.sync_copy(obuf, hist.at[pl.ds(sid * STRIPE + DUMP, DUMP)])

    pltpu.sync_copy(mm_ref, mmbuf)
    xmin = mmbuf[pl.ds(0, 16)]
    ymin = mmbuf[pl.ds(128, 16)]
    zmin = mmbuf[pl.ds(256, 16)]
    xmax = mmbuf[pl.ds(512, 16)]
    ymax = mmbuf[pl.ds(640, 16)]
    zmax = mmbuf[pl.ds(768, 16)]
    rx = xmax - xmin
    ry = ymax - ymin
    rz = zmax - zmin

    def init_ones(t, _):
        ones[pl.ds(t * 16, 16)] = jnp.full((16,), 1.0, jnp.float32)
        return 0
    lax.fori_loop(0, CHUNK // 16, init_ones, 0)

    plsc.subcore_barrier()

    dummy = out_ref.at[cid, 0, pl.ds(0, CHUNK)]

    def compute_chunk(pcb, idxs, idxc, zss):
        def row_body(j, _):
            o128 = j * 128
            for u in range(8):
                xv = pcb[j, 0, pl.ds(u * 16, 16)]
                yv = pcb[j, 1, pl.ds(u * 16, 16)]
                zv = pcb[j, 2, pl.ds(u * 16, 16)]
                xb = ((xv - xmin) * (WIDTH - 1.0) / rx).astype(jnp.int32)
                yb = ((yv - ymin) * (HEIGHT - 1.0) / ry).astype(jnp.int32)
                idx = (511 - yb) * 512 + xb
                idx = jnp.minimum(jnp.maximum(idx, 0), NBINS - 1)
                zs = (zv - zmin) * INTENSITY / rz
                o = o128 + u * 16
                idxs[pl.ds(o, 16)] = idx
                idxc[pl.ds(o, 16)] = idx + NBINS
                zss[pl.ds(o, 16)] = zs
            return 0
        lax.fori_loop(0, KROWS, row_body, 0)

    def pair_body(p, _):
        g0 = wid + (2 * p) * NW
        g1 = g0 + NW

        @pl.when(g0 < NCHUNKS)
        def _():
            pltpu.async_copy(pc_ref.at[pl.ds(g0 * KROWS, KROWS)], pcbuf, semia)

        @pl.when(g1 < NCHUNKS)
        def _():
            pltpu.async_copy(pc_ref.at[pl.ds(g1 * KROWS, KROWS)], pcbufb,
                             semib)

        @pl.when(g0 < NCHUNKS)
        def _():
            pltpu.make_async_copy(pc_ref.at[pl.ds(g0 * KROWS, KROWS)],
                                  pcbuf, semia).wait()

            @pl.when(p > 0)
            def _():
                pltpu.make_async_copy(dummy, zssa, sema).wait()
                pltpu.make_async_copy(dummy, zssa, sema).wait()
            compute_chunk(pcbuf, idxsa, idxca, zssa)
            pltpu.async_copy(zssa, hist.at[idxsa], sema, add=True)
            pltpu.async_copy(ones, hist.at[idxca], sema, add=True)

        @pl.when(g1 < NCHUNKS)
        def _():
            pltpu.make_async_copy(pc_ref.at[pl.ds(g1 * KROWS, KROWS)],
                                  pcbufb, semib).wait()

            @pl.when(p > 0)
            def _():
                pltpu.make_async_copy(dummy, zssb, semb).wait()
                pltpu.make_async_copy(dummy, zssb, semb).wait()
            compute_chunk(pcbufb, idxsb, idxcb, zssb)
            pltpu.async_copy(zssb, hist.at[idxsb], semb, add=True)
            pltpu.async_copy(ones, hist.at[idxcb], semb, add=True)
        return 0
    lax.fori_loop(0, NPAIRS, pair_body, 0)

    pltpu.make_async_copy(dummy, zssa, sema).wait()
    pltpu.make_async_copy(dummy, zssa, sema).wait()
    pltpu.make_async_copy(dummy, zssb, semb).wait()
    pltpu.make_async_copy(dummy, zssb, semb).wait()

    plsc.subcore_barrier()

    pltpu.sync_copy(hist.at[pl.ds(sid * DUMP, DUMP)], obuf)
    pltpu.sync_copy(obuf, out_ref.at[cid, 0, pl.ds(sid * DUMP, DUMP)])
    pltpu.sync_copy(hist.at[pl.ds(NBINS + sid * DUMP, DUMP)], obuf)
    pltpu.sync_copy(obuf, out_ref.at[cid, 1, pl.ds(sid * DUMP, DUMP)])


_sc_call = pl.kernel(
    _sc_body,
    out_type=jax.ShapeDtypeStruct((NC, 2, NBINS), jnp.float32),
    mesh=plsc.VectorSubcoreMesh(core_axis_name="c", subcore_axis_name="s",
                                num_cores=NC, num_subcores=NS),
    scratch_types=[
        pltpu.VMEM((KROWS, 8, 128), jnp.float32),
        pltpu.VMEM((KROWS, 8, 128), jnp.float32),
        pltpu.VMEM((1024,), jnp.float32),
        pltpu.VMEM((CHUNK,), jnp.int32),
        pltpu.VMEM((CHUNK,), jnp.int32),
        pltpu.VMEM((CHUNK,), jnp.float32),
        pltpu.VMEM((CHUNK,), jnp.int32),
        pltpu.VMEM((CHUNK,), jnp.int32),
        pltpu.VMEM((CHUNK,), jnp.float32),
        pltpu.VMEM((CHUNK,), jnp.float32),
        pltpu.VMEM((DUMP,), jnp.float32),
        pltpu.VMEM_SHARED((HIST_WORDS,), jnp.float32),
        pltpu.SemaphoreType.DMA,
        pltpu.SemaphoreType.DMA,
        pltpu.SemaphoreType.DMA,
        pltpu.SemaphoreType.DMA,
    ],
    compiler_params=pltpu.CompilerParams(needs_layout_passes=False),
)


@jax.jit
def kernel(pc):
    mm, pcf = _minmax_call(pc.T)
    parts = _sc_call(pcf, mm.reshape(-1))
    return _combine_call(parts.reshape(NC, 2, HEIGHT, WIDTH))


# speedup vs baseline: 1.1624x; 1.0143x over previous
import functools

import jax
import jax.numpy as jnp
from jax import lax
from jax.experimental import pallas as pl
from jax.experimental.pallas import tpu as pltpu
from jax.experimental.pallas import tpu_sc as plsc

HEIGHT = 512
WIDTH = 512
INTENSITY = 255.0
NBINS = HEIGHT * WIDTH
HIST_WORDS = 2 * NBINS

N_POINTS = 2_000_000
CHUNK = 3200
KROWS = CHUNK // 128
NCHUNKS = N_POINTS // CHUNK
NC, NS = 2, 16
NW = NC * NS

MM_BLOCK = 40_960
MM_GRID = -(-N_POINTS // MM_BLOCK)

STRIPE = HIST_WORDS // NS
DUMP = NBINS // NS


def _minmax_body(pc_ref, o_ref, fl_ref):
    i = pl.program_id(0)
    d = pc_ref[:]
    d8 = jnp.concatenate(
        [d, jnp.zeros((5, MM_BLOCK), jnp.float32)], axis=0)
    fl_ref[:] = jnp.transpose(
        d8.reshape(8, MM_BLOCK // 128, 128), (1, 0, 2))
    cid = lax.broadcasted_iota(jnp.int32, (3, MM_BLOCK), 1)
    valid = (i * MM_BLOCK + cid) < N_POINTS
    dlo = jnp.where(valid, d, jnp.inf)
    dhi = jnp.where(valid, d, -jnp.inf)
    dmin = jnp.broadcast_to(jnp.min(dlo, axis=1, keepdims=True), (3, 128))
    dmax = jnp.broadcast_to(jnp.max(dhi, axis=1, keepdims=True), (3, 128))
    pad1 = jnp.zeros((1, 128), jnp.float32)
    cur = jnp.concatenate([dmin, pad1, dmax, pad1], axis=0)
    rows = lax.broadcasted_iota(jnp.int32, (8, 128), 0)
    acc = o_ref[:]
    comb = jnp.where(rows < 4, jnp.minimum(acc, cur), jnp.maximum(acc, cur))
    o_ref[:] = jnp.where(i == 0, cur, comb)


_minmax_call = pl.pallas_call(
    _minmax_body,
    grid=(MM_GRID,),
    in_specs=[pl.BlockSpec((3, MM_BLOCK), lambda i: (0, i))],
    out_specs=[pl.BlockSpec((8, 128), lambda i: (0, 0)),
               pl.BlockSpec((MM_BLOCK // 128, 8, 128), lambda i: (i, 0, 0))],
    out_shape=[jax.ShapeDtypeStruct((8, 128), jnp.float32),
               jax.ShapeDtypeStruct(
                   (MM_GRID * MM_BLOCK // 128, 8, 128), jnp.float32)],
)


def _combine_body(p_ref, o_ref):
    s = p_ref[0, 0] + p_ref[1, 0]
    c = p_ref[0, 1] + p_ref[1, 1]
    o_ref[:] = jnp.where(c > 0, s / c, 0.0)


_combine_call = pl.pallas_call(
    _combine_body,
    grid=(8,),
    in_specs=[pl.BlockSpec((2, 2, 64, 512), lambda r: (0, 0, r, 0))],
    out_specs=pl.BlockSpec((64, 512), lambda r: (r, 0)),
    out_shape=jax.ShapeDtypeStruct((HEIGHT, WIDTH), jnp.float32),
)


NPAIRS = (NCHUNKS // NW + 1 + 1) // 2


def _sc_body(pc_ref, mm_ref, out_ref,
             pcbuf, pcbufb, mmbuf, idxsa, zssa, idxsb, zssb,
             ones, obuf, hsum, hcnt, sema, semb, semia, semib):
    cid = lax.axis_index("c")
    sid = lax.axis_index("s")
    wid = sid * NC + cid

    def zero_obuf(t, _):
        obuf[pl.ds(t * 16, 16)] = jnp.zeros((16,), jnp.float32)
        return 0
    lax.fori_loop(0, DUMP // 16, zero_obuf, 0)
    pltpu.sync_copy(obuf, hsum.at[pl.ds(sid * DUMP, DUMP)])
    pltpu.sync_copy(obuf, hcnt.at[pl.ds(sid * DUMP, DUMP)])

    pltpu.sync_copy(mm_ref, mmbuf)
    xmin = mmbuf[pl.ds(0, 16)]
    ymin = mmbuf[pl.ds(128, 16)]
    zmin = mmbuf[pl.ds(256, 16)]
    xmax = mmbuf[pl.ds(512, 16)]
    ymax = mmbuf[pl.ds(640, 16)]
    zmax = mmbuf[pl.ds(768, 16)]
    rx = xmax - xmin
    ry = ymax - ymin
    rz = zmax - zmin

    def init_ones(t, _):
        ones[pl.ds(t * 16, 16)] = jnp.full((16,), 1.0, jnp.float32)
        return 0
    lax.fori_loop(0, CHUNK // 16, init_ones, 0)

    plsc.subcore_barrier()

    dummy = out_ref.at[cid, 0, pl.ds(0, CHUNK)]

    def compute_chunk(pcb, idxs, zss):
        def row_body(j, _):
            o128 = j * 128
            for u in range(8):
                xv = pcb[j, 0, pl.ds(u * 16, 16)]
                yv = pcb[j, 1, pl.ds(u * 16, 16)]
                zv = pcb[j, 2, pl.ds(u * 16, 16)]
                xb = ((xv - xmin) * (WIDTH - 1.0) / rx).astype(jnp.int32)
                yb = ((yv - ymin) * (HEIGHT - 1.0) / ry).astype(jnp.int32)
                idx = (511 - yb) * 512 + xb
                idx = jnp.minimum(jnp.maximum(idx, 0), NBINS - 1)
                zs = (zv - zmin) * INTENSITY / rz
                o = o128 + u * 16
                idxs[pl.ds(o, 16)] = idx
                zss[pl.ds(o, 16)] = zs
            return 0
        lax.fori_loop(0, KROWS, row_body, 0)

    def pair_body(p, _):
        g0 = wid + (2 * p) * NW
        g1 = g0 + NW

        @pl.when(g0 < NCHUNKS)
        def _():
            pltpu.async_copy(pc_ref.at[pl.ds(g0 * KROWS, KROWS)], pcbuf, semia)

        @pl.when(g1 < NCHUNKS)
        def _():
            pltpu.async_copy(pc_ref.at[pl.ds(g1 * KROWS, KROWS)], pcbufb,
                             semib)

        @pl.when(g0 < NCHUNKS)
        def _():
            pltpu.make_async_copy(pc_ref.at[pl.ds(g0 * KROWS, KROWS)],
                                  pcbuf, semia).wait()

            @pl.when(p > 0)
            def _():
                pltpu.make_async_copy(dummy, zssa, sema).wait()
                pltpu.make_async_copy(dummy, zssa, sema).wait()
            compute_chunk(pcbuf, idxsa, zssa)
            pltpu.async_copy(zssa, hsum.at[idxsa], sema, add=True)
            pltpu.async_copy(ones, hcnt.at[idxsa], sema, add=True)

        @pl.when(g1 < NCHUNKS)
        def _():
            pltpu.make_async_copy(pc_ref.at[pl.ds(g1 * KROWS, KROWS)],
                                  pcbufb, semib).wait()

            @pl.when(p > 0)
            def _():
                pltpu.make_async_copy(dummy, zssb, semb).wait()
                pltpu.make_async_copy(dummy, zssb, semb).wait()
            compute_chunk(pcbufb, idxsb, zssb)
            pltpu.async_copy(zssb, hsum.at[idxsb], semb, add=True)
            pltpu.async_copy(ones, hcnt.at[idxsb], semb, add=True)
        return 0
    lax.fori_loop(0, NPAIRS, pair_body, 0)

    pltpu.make_async_copy(dummy, zssa, sema).wait()
    pltpu.make_async_copy(dummy, zssa, sema).wait()
    pltpu.make_async_copy(dummy, zssb, semb).wait()
    pltpu.make_async_copy(dummy, zssb, semb).wait()

    plsc.subcore_barrier()

    pltpu.sync_copy(hsum.at[pl.ds(sid * DUMP, DUMP)], obuf)
    pltpu.sync_copy(obuf, out_ref.at[cid, 0, pl.ds(sid * DUMP, DUMP)])
    pltpu.sync_copy(hcnt.at[pl.ds(sid * DUMP, DUMP)], obuf)
    pltpu.sync_copy(obuf, out_ref.at[cid, 1, pl.ds(sid * DUMP, DUMP)])


_sc_call = pl.kernel(
    _sc_body,
    out_type=jax.ShapeDtypeStruct((NC, 2, NBINS), jnp.float32),
    mesh=plsc.VectorSubcoreMesh(core_axis_name="c", subcore_axis_name="s",
                                num_cores=NC, num_subcores=NS),
    scratch_types=[
        pltpu.VMEM((KROWS, 8, 128), jnp.float32),
        pltpu.VMEM((KROWS, 8, 128), jnp.float32),
        pltpu.VMEM((1024,), jnp.float32),
        pltpu.VMEM((CHUNK,), jnp.int32),
        pltpu.VMEM((CHUNK,), jnp.float32),
        pltpu.VMEM((CHUNK,), jnp.int32),
        pltpu.VMEM((CHUNK,), jnp.float32),
        pltpu.VMEM((CHUNK,), jnp.float32),
        pltpu.VMEM((DUMP,), jnp.float32),
        pltpu.VMEM_SHARED((NBINS,), jnp.float32),
        pltpu.VMEM_SHARED((NBINS,), jnp.float32),
        pltpu.SemaphoreType.DMA,
        pltpu.SemaphoreType.DMA,
        pltpu.SemaphoreType.DMA,
        pltpu.SemaphoreType.DMA,
    ],
    compiler_params=pltpu.CompilerParams(needs_layout_passes=False),
)


@jax.jit
def kernel(pc):
    mm, pcf = _minmax_call(pc.T)
    parts = _sc_call(pcf, mm.reshape(-1))
    return _combine_call(parts.reshape(NC, 2, HEIGHT, WIDTH))
